# baseline (speedup 1.0000x reference)
import functools

import jax
import jax.numpy as jnp
from jax import lax
from jax.experimental import pallas as pl
from jax.experimental.pallas import tpu as pltpu
from jax.experimental.pallas import tpu_sc as plsc

_B = 16384
_D = 128
_NC = 2
_NS = 16
_NW = _NC * _NS
_BPW = _B // _NW
_CH = 128
_NCHUNK = _BPW // _CH
_LANES = 16
_UNROLL = _D // _LANES
_SCALE = 0.003 / (2.0 * _B * _D)

_mesh = plsc.VectorSubcoreMesh(core_axis_name="c", subcore_axis_name="s")


@functools.partial(
    pl.kernel,
    mesh=_mesh,
    out_type=[
        jax.ShapeDtypeStruct((_NS, _LANES), jnp.float32),
        jax.ShapeDtypeStruct((_NS, _LANES), jnp.float32),
    ],
    scratch_types=[
        pltpu.VMEM((_NCHUNK, _CH), jnp.int32),
        pltpu.VMEM((2, _CH, _D), jnp.float32),
        pltpu.VMEM((2, _CH, _D), jnp.float32),
        pltpu.VMEM((_LANES,), jnp.float32),
        pltpu.SemaphoreType.DMA,
        pltpu.SemaphoreType.DMA,
        pltpu.SemaphoreType.DMA,
        pltpu.SemaphoreType.DMA,
    ],
)
def _center_loss_sc(feat_hbm, lbl_hbm, cent_hbm, out0_hbm, out1_hbm,
                    idx_v, feat_v, rows_v, part_v,
                    sg0, sg1, sf0, sf1):
    c = lax.axis_index("c")
    s = lax.axis_index("s")
    w = s * _NC + c

    pltpu.sync_copy(lbl_hbm.at[pl.ds(w * _NCHUNK, _NCHUNK)], idx_v)

    gsems = (sg0, sg1)
    fsems = (sf0, sf1)

    def start(j, slot):
        g = pltpu.async_copy(cent_hbm.at[idx_v.at[j]], rows_v.at[slot], gsems[slot])
        f = pltpu.async_copy(
            feat_hbm.at[pl.ds(w * _BPW + j * _CH, _CH)], feat_v.at[slot], fsems[slot]
        )
        return g, f

    pending = start(0, 0)
    accs = tuple(jnp.zeros((_LANES,), jnp.float32) for _ in range(_UNROLL))

    for j in range(_NCHUNK):
        slot = j % 2
        nxt = start(j + 1, (j + 1) % 2) if j + 1 < _NCHUNK else None
        pending[0].wait()
        pending[1].wait()
        fbuf = feat_v.at[slot]
        cbuf = rows_v.at[slot]

        def body(i, acc, fbuf=fbuf, cbuf=cbuf):
            out = []
            for u in range(_UNROLL):
                fv = fbuf[i, pl.ds(u * _LANES, _LANES)]
                cv = cbuf[i, pl.ds(u * _LANES, _LANES)]
                d = fv - cv
                out.append(acc[u] + d * d)
            return tuple(out)

        accs = lax.fori_loop(0, _CH, body, accs)
        pending = nxt

    total = accs[0]
    for u in range(1, _UNROLL):
        total = total + accs[u]
    part_v[...] = total * _SCALE

    @pl.when(c == 0)
    def _():
        pltpu.sync_copy(part_v, out0_hbm.at[s])

    @pl.when(c == 1)
    def _():
        pltpu.sync_copy(part_v, out1_hbm.at[s])


def kernel(features, labels, centers):
    lbl = labels.reshape(-1).astype(jnp.int32).reshape(_B // _D, _D)
    out0, out1 = _center_loss_sc(features, lbl, centers)
    return jnp.sum(out0) + jnp.sum(out1)


# speedup vs baseline: 1.1228x; 1.1228x over previous
import functools

import jax
import jax.numpy as jnp
from jax import lax
from jax.experimental import pallas as pl
from jax.experimental.pallas import tpu as pltpu
from jax.experimental.pallas import tpu_sc as plsc

_B = 16384
_D = 128
_NC = 2
_NS = 16
_NW = _NC * _NS
_BPW = _B // _NW
_CH = 128
_NCHUNK = _BPW // _CH
_LANES = 16
_UNROLL = _D // _LANES
_ROWS_PER_IT = 2
_SCALE = 0.003 / (2.0 * _B * _D)

_mesh = plsc.VectorSubcoreMesh(core_axis_name="c", subcore_axis_name="s")


@functools.partial(
    pl.kernel,
    mesh=_mesh,
    out_type=jax.ShapeDtypeStruct((_NW, _LANES), jnp.float32),
    scratch_types=[
        pltpu.VMEM((_NCHUNK, _CH), jnp.int32),
        pltpu.VMEM((2, _CH, _D), jnp.float32),
        pltpu.VMEM((2, _CH, _D), jnp.float32),
        pltpu.VMEM((_LANES,), jnp.float32),
        pltpu.SemaphoreType.DMA,
        pltpu.SemaphoreType.DMA,
        pltpu.SemaphoreType.DMA,
        pltpu.SemaphoreType.DMA,
    ],
)
def _center_loss_sc(feat_hbm, lbl_hbm, cent_hbm, out_hbm,
                    idx_v, feat_v, rows_v, part_v,
                    sg0, sg1, sf0, sf1):
    c = lax.axis_index("c")
    s = lax.axis_index("s")
    w = s * _NC + c

    gsems = (sg0, sg1)
    fsems = (sf0, sf1)

    f0 = pltpu.async_copy(feat_hbm.at[pl.ds(w * _BPW, _CH)], feat_v.at[0], fsems[0])
    pltpu.sync_copy(lbl_hbm.at[pl.ds(w * _NCHUNK, _NCHUNK)], idx_v)
    g0 = pltpu.async_copy(cent_hbm.at[idx_v.at[0]], rows_v.at[0], gsems[0])
    pending = (g0, f0)

    def start(j, slot):
        g = pltpu.async_copy(cent_hbm.at[idx_v.at[j]], rows_v.at[slot], gsems[slot])
        f = pltpu.async_copy(
            feat_hbm.at[pl.ds(w * _BPW + j * _CH, _CH)], feat_v.at[slot], fsems[slot]
        )
        return g, f

    accs = tuple(jnp.zeros((_LANES,), jnp.float32) for _ in range(_UNROLL))

    for j in range(_NCHUNK):
        slot = j % 2
        nxt = start(j + 1, (j + 1) % 2) if j + 1 < _NCHUNK else None
        pending[0].wait()
        pending[1].wait()
        fbuf = feat_v.at[slot]
        cbuf = rows_v.at[slot]

        def body(i, acc, fbuf=fbuf, cbuf=cbuf):
            out = list(acc)
            for r in range(_ROWS_PER_IT):
                row = i * _ROWS_PER_IT + r
                for u in range(_UNROLL):
                    fv = fbuf[row, pl.ds(u * _LANES, _LANES)]
                    cv = cbuf[row, pl.ds(u * _LANES, _LANES)]
                    d = fv - cv
                    out[u] = out[u] + d * d
            return tuple(out)

        accs = lax.fori_loop(0, _CH // _ROWS_PER_IT, body, accs)
        pending = nxt

    total = accs[0]
    for u in range(1, _UNROLL):
        total = total + accs[u]
    part_v[...] = total * _SCALE

    pltpu.sync_copy(part_v, out_hbm.at[w])


def kernel(features, labels, centers):
    lbl = labels.reshape(-1).astype(jnp.int32).reshape(_B // _D, _D)
    out = _center_loss_sc(features, lbl, centers)
    return jnp.sum(out)
